# Initial kernel scaffold; baseline (speedup 1.0000x reference)
import jax
import jax.numpy as jnp
from jax.experimental import pallas as pl


def kernel(x, w_ih0, w_hh0, b_ih0, b_hh0, w_ih1, w_hh1, b_ih1, b_hh1, w_fc, b_fc):
    raise NotImplementedError("write your pallas kernel here")


# speedup vs baseline: 3.1948x; 3.1948x over previous
import jax
import jax.numpy as jnp
from jax.experimental import pallas as pl
from jax.experimental.pallas import tpu as pltpu

H = 50
HP = 56
G = 4 * HP


def _lstm_fused_kernel(x_ref, wi0_ref, wh0_ref, b0_ref, wi1_ref, wh1_ref,
                       b1_ref, wfc_ref, bfc_ref, out_ref,
                       h1_ref, c1_ref, h2_ref, c2_ref):
    tc = pl.program_id(1)
    num_tc = pl.num_programs(1)

    @pl.when(tc == 0)
    def _init():
        h1_ref[...] = jnp.zeros_like(h1_ref)
        c1_ref[...] = jnp.zeros_like(c1_ref)
        h2_ref[...] = jnp.zeros_like(h2_ref)
        c2_ref[...] = jnp.zeros_like(c2_ref)

    wi0 = wi0_ref[...]
    wh0 = wh0_ref[...]
    b0 = b0_ref[...]
    wi1 = wi1_ref[...]
    wh1 = wh1_ref[...]
    b1 = b1_ref[...]

    tt = x_ref.shape[0]

    def step(t, carry):
        xt = x_ref[t]
        g0 = jax.lax.dot_general(wi0, xt, (((1,), (1,)), ((), ())),
                                 preferred_element_type=jnp.float32)
        g0 = g0 + jnp.dot(wh0, h1_ref[...],
                          preferred_element_type=jnp.float32) + b0
        i0 = jax.nn.sigmoid(g0[0:HP])
        f0 = jax.nn.sigmoid(g0[HP:2 * HP])
        u0 = jnp.tanh(g0[2 * HP:3 * HP])
        o0 = jax.nn.sigmoid(g0[3 * HP:4 * HP])
        c1 = f0 * c1_ref[...] + i0 * u0
        h1 = o0 * jnp.tanh(c1)
        c1_ref[...] = c1
        h1_ref[...] = h1

        g1 = (jnp.dot(wi1, h1, preferred_element_type=jnp.float32)
              + jnp.dot(wh1, h2_ref[...], preferred_element_type=jnp.float32)
              + b1)
        i1 = jax.nn.sigmoid(g1[0:HP])
        f1 = jax.nn.sigmoid(g1[HP:2 * HP])
        u1 = jnp.tanh(g1[2 * HP:3 * HP])
        o1 = jax.nn.sigmoid(g1[3 * HP:4 * HP])
        c2 = f1 * c2_ref[...] + i1 * u1
        h2 = o1 * jnp.tanh(c2)
        c2_ref[...] = c2
        h2_ref[...] = h2
        return 0

    jax.lax.fori_loop(0, tt, step, 0)

    @pl.when(tc == num_tc - 1)
    def _final():
        out_ref[...] = jnp.dot(wfc_ref[...], h2_ref[...],
                               preferred_element_type=jnp.float32) + bfc_ref[...]


def _pad_gate_rows(w):
    k = w.shape[1]
    return jnp.pad(w.reshape(4, H, k), ((0, 0), (0, HP - H), (0, 0))).reshape(G, k)


def kernel(x, w_ih0, w_hh0, b_ih0, b_hh0, w_ih1, w_hh1, b_ih1, b_hh1,
           w_fc, b_fc):
    B, T = x.shape[0], x.shape[1]
    x2 = x.reshape(B, T, -1)
    I = x2.shape[-1]
    xT = jnp.swapaxes(x2, 0, 1)

    NB = 2
    Bb = B // NB
    TT = 32
    TC = T // TT

    wi0 = _pad_gate_rows(w_ih0)
    wh0 = _pad_gate_rows(jnp.pad(w_hh0, ((0, 0), (0, HP - H))))
    b0 = jnp.pad((b_ih0 + b_hh0).reshape(4, H),
                 ((0, 0), (0, HP - H))).reshape(G, 1)
    wi1 = _pad_gate_rows(jnp.pad(w_ih1, ((0, 0), (0, HP - H))))
    wh1 = _pad_gate_rows(jnp.pad(w_hh1, ((0, 0), (0, HP - H))))
    b1 = jnp.pad((b_ih1 + b_hh1).reshape(4, H),
                 ((0, 0), (0, HP - H))).reshape(G, 1)
    wfc = jnp.pad(w_fc, ((0, 0), (0, HP - H)))
    O = wfc.shape[0]
    bfc = b_fc.reshape(O, 1)

    full = lambda a: pl.BlockSpec(a.shape, lambda b, t: (0,) * a.ndim)

    outT = pl.pallas_call(
        _lstm_fused_kernel,
        out_shape=jax.ShapeDtypeStruct((O, B), jnp.float32),
        grid=(NB, TC),
        in_specs=[
            pl.BlockSpec((TT, Bb, I), lambda b, t: (t, b, 0)),
            full(wi0), full(wh0), full(b0),
            full(wi1), full(wh1), full(b1),
            full(wfc), full(bfc),
        ],
        out_specs=pl.BlockSpec((O, Bb), lambda b, t: (0, b)),
        scratch_shapes=[
            pltpu.VMEM((HP, Bb), jnp.float32),
            pltpu.VMEM((HP, Bb), jnp.float32),
            pltpu.VMEM((HP, Bb), jnp.float32),
            pltpu.VMEM((HP, Bb), jnp.float32),
        ],
        compiler_params=pltpu.CompilerParams(
            dimension_semantics=("parallel", "arbitrary"),
        ),
        name="lstm2_fused",
    )(xT, wi0, wh0, b0, wi1, wh1, b1, wfc, bfc)

    return outT.T.reshape(-1, 16, 9)


# speedup vs baseline: 4.2073x; 1.3169x over previous
import jax
import jax.numpy as jnp
from jax.experimental import pallas as pl
from jax.experimental.pallas import tpu as pltpu

H = 50
HP = 56
G = 4 * HP


def _gates(g):
    i = jax.nn.sigmoid(g[0:HP])
    f = jax.nn.sigmoid(g[HP:2 * HP])
    u = jnp.tanh(g[2 * HP:3 * HP])
    o = jax.nn.sigmoid(g[3 * HP:4 * HP])
    return i, f, u, o


def _lstm_fused_kernel(x_ref, wi0_ref, wh0_ref, b0_ref, wi1_ref, wh1_ref,
                       b1_ref, wfc_ref, bfc_ref, out_ref,
                       h1_ref, c1_ref, h2_ref, c2_ref):
    tc = pl.program_id(1)
    num_tc = pl.num_programs(1)
    tt = x_ref.shape[0]
    Bb = x_ref.shape[2]

    def layer0_step(xt, h1, c1):
        g0 = (jnp.dot(wi0_ref[...], xt, preferred_element_type=jnp.float32)
              + jnp.dot(wh0_ref[...], h1, preferred_element_type=jnp.float32)
              + b0_ref[...])
        i0, f0, u0, o0 = _gates(g0)
        c1n = f0 * c1 + i0 * u0
        h1n = o0 * jnp.tanh(c1n)
        return h1n, c1n

    def layer1_step(h1, h2, c2):
        g1 = (jnp.dot(wi1_ref[...], h1, preferred_element_type=jnp.float32)
              + jnp.dot(wh1_ref[...], h2, preferred_element_type=jnp.float32)
              + b1_ref[...])
        i1, f1, u1, o1 = _gates(g1)
        c2n = f1 * c2 + i1 * u1
        h2n = o1 * jnp.tanh(c2n)
        return h2n, c2n

    @pl.when(tc == 0)
    def _init():
        z = jnp.zeros((HP, Bb), jnp.float32)
        h1n, c1n = layer0_step(x_ref[0], z, z)
        h1_ref[...] = h1n
        c1_ref[...] = c1n
        h2_ref[...] = z
        c2_ref[...] = z

    def body(k, carry):
        h1 = h1_ref[...]
        h2n, c2n = layer1_step(h1, h2_ref[...], c2_ref[...])
        h2_ref[...] = h2n
        c2_ref[...] = c2n
        h1n, c1n = layer0_step(x_ref[k], h1, c1_ref[...])
        h1_ref[...] = h1n
        c1_ref[...] = c1n
        return 0

    start = jnp.where(tc == 0, 1, 0)
    jax.lax.fori_loop(start, tt, body, 0)

    @pl.when(tc == num_tc - 1)
    def _final():
        h2n, _ = layer1_step(h1_ref[...], h2_ref[...], c2_ref[...])
        out_ref[...] = jnp.dot(wfc_ref[...], h2n,
                               preferred_element_type=jnp.float32) + bfc_ref[...]


def _pad_gate_rows(w):
    k = w.shape[1]
    return jnp.pad(w.reshape(4, H, k), ((0, 0), (0, HP - H), (0, 0))).reshape(G, k)


def kernel(x, w_ih0, w_hh0, b_ih0, b_hh0, w_ih1, w_hh1, b_ih1, b_hh1,
           w_fc, b_fc):
    B, T = x.shape[0], x.shape[1]
    x2 = x.reshape(B, T, -1)
    I = x2.shape[-1]
    xT = jnp.transpose(x2, (1, 2, 0))

    NB = 2
    Bb = B // NB
    TT = 32
    TC = T // TT

    wi0 = _pad_gate_rows(w_ih0)
    wh0 = _pad_gate_rows(jnp.pad(w_hh0, ((0, 0), (0, HP - H))))
    b0 = jnp.pad((b_ih0 + b_hh0).reshape(4, H),
                 ((0, 0), (0, HP - H))).reshape(G, 1)
    wi1 = _pad_gate_rows(jnp.pad(w_ih1, ((0, 0), (0, HP - H))))
    wh1 = _pad_gate_rows(jnp.pad(w_hh1, ((0, 0), (0, HP - H))))
    b1 = jnp.pad((b_ih1 + b_hh1).reshape(4, H),
                 ((0, 0), (0, HP - H))).reshape(G, 1)
    wfc = jnp.pad(w_fc, ((0, 0), (0, HP - H)))
    O = wfc.shape[0]
    bfc = b_fc.reshape(O, 1)

    full = lambda a: pl.BlockSpec(a.shape, lambda b, t: (0,) * a.ndim)

    outT = pl.pallas_call(
        _lstm_fused_kernel,
        out_shape=jax.ShapeDtypeStruct((O, B), jnp.float32),
        grid=(NB, TC),
        in_specs=[
            pl.BlockSpec((TT, I, Bb), lambda b, t: (t, 0, b)),
            full(wi0), full(wh0), full(b0),
            full(wi1), full(wh1), full(b1),
            full(wfc), full(bfc),
        ],
        out_specs=pl.BlockSpec((O, Bb), lambda b, t: (0, b)),
        scratch_shapes=[
            pltpu.VMEM((HP, Bb), jnp.float32),
            pltpu.VMEM((HP, Bb), jnp.float32),
            pltpu.VMEM((HP, Bb), jnp.float32),
            pltpu.VMEM((HP, Bb), jnp.float32),
        ],
        compiler_params=pltpu.CompilerParams(
            dimension_semantics=("parallel", "arbitrary"),
        ),
        name="lstm2_fused",
    )(xT, wi0, wh0, b0, wi1, wh1, b1, wfc, bfc)

    return outT.T.reshape(-1, 16, 9)
